# baseline (speedup 1.0000x reference)
import functools

import jax
import jax.numpy as jnp
from jax import lax
from jax.experimental import pallas as pl
from jax.experimental.pallas import tpu as pltpu
from jax.experimental.pallas import tpu_sc as plsc

B = 16384
N_COLS = 26
VOCAB = 100000
DIM = 16

NUM_CORES = 2
NUM_SUBCORES = 16
NW = NUM_CORES * NUM_SUBCORES
ROWS = B * N_COLS
RPW = ROWS // NW
G = 128
NSUB = 8
CHUNK = NSUB * G
NCHUNK = RPW // CHUNK


def _gather_body(idx_hbm, tab_hbm, out_hbm, idx_v, rows_v, sem):
  wid = lax.axis_index("s") * NUM_CORES + lax.axis_index("c")
  base = wid * RPW

  def chunk_body(k, carry):
    start = pl.multiple_of(base + k * CHUNK, CHUNK)
    r0 = pl.multiple_of(start // G, NSUB)
    pltpu.sync_copy(idx_hbm.at[pl.ds(r0, NSUB)], idx_v)
    copies = []
    for j in range(NSUB):
      copies.append(
          pltpu.async_copy(
              tab_hbm.at[idx_v.at[j]], rows_v.at[pl.ds(j * G, G)], sem))
    for c in copies:
      c.wait()
    pltpu.sync_copy(rows_v, out_hbm.at[pl.ds(start, CHUNK)])
    return carry

  lax.fori_loop(0, NCHUNK, chunk_body, 0)


@functools.partial(jax.jit, static_argnames=())
def _embed(flat_idx, flat_tab):
  mesh = plsc.VectorSubcoreMesh(core_axis_name="c", subcore_axis_name="s")
  f = pl.kernel(
      _gather_body,
      mesh=mesh,
      out_type=jax.ShapeDtypeStruct((ROWS, DIM), jnp.float32),
      scratch_types=[
          pltpu.VMEM((NSUB, G), jnp.int32),
          pltpu.VMEM((CHUNK, DIM), jnp.float32),
          pltpu.SemaphoreType.DMA,
      ],
      compiler_params=pltpu.CompilerParams(use_tc_tiling_on_sc=False),
  )
  return f(flat_idx, flat_tab)


def kernel(value, tables):
  flat_tab = tables.reshape(N_COLS * VOCAB, DIM)
  offs = (jnp.arange(N_COLS, dtype=jnp.int32) * VOCAB)[None, :]
  flat_idx = (value.astype(jnp.int32) + offs).reshape(ROWS // G, G)
  out = _embed(flat_idx, flat_tab)
  return out.reshape(B, N_COLS, DIM)


# speedup vs baseline: 1.2038x; 1.2038x over previous
import functools

import jax
import jax.numpy as jnp
from jax import lax
from jax.experimental import pallas as pl
from jax.experimental.pallas import tpu as pltpu
from jax.experimental.pallas import tpu_sc as plsc

B = 16384
N_COLS = 26
VOCAB = 100000
DIM = 16

NUM_CORES = 2
NUM_SUBCORES = 16
NW = NUM_CORES * NUM_SUBCORES
BT = B // 128
NTASK = N_COLS * BT
TPW = NTASK // NW
GPC = 8
CHUNK = GPC * 128
NCHUNK = TPW // GPC


def _gather_body(idx_hbm, tab_hbm, out_hbm, idx_v, rows_v, tiles_v, pat_v, sem):
  wid = lax.axis_index("s") * NUM_CORES + lax.axis_index("c")
  t0 = wid * TPW

  def pat_body(l, carry):
    v = lax.iota(jnp.int32, 16) + l * 16
    pat_v[pl.ds(pl.multiple_of(l * 16, 16), 16)] = v
    return carry
  lax.fori_loop(0, GPC, pat_body, 0)

  def chunk_body(k, carry):
    t = t0 + k * GPC
    c = t // BT
    bt0 = pl.multiple_of(t - c * BT, GPC)
    pltpu.sync_copy(idx_hbm.at[c].at[pl.ds(bt0, GPC)], idx_v)
    copies = []
    for j in range(GPC):
      copies.append(
          pltpu.async_copy(
              tab_hbm.at[idx_v.at[j]], rows_v.at[pl.ds(j * 128, 128)], sem))
    for cp in copies:
      cp.wait()
    for d in range(DIM):
      dt, di = d // 8, d % 8
      def tr_body(jl, carry, _dt=dt, _di=di, _d=d):
        j = jl // GPC
        l = jl - j * GPC
        off = pl.multiple_of(l * 16, 16)
        ridx = pat_v[pl.ds(off, 16)] + j * 128
        vals = plsc.load_gather(rows_v, [ridx, jnp.full((16,), _d, jnp.int32)])
        tiles_v[_dt, j, _di, pl.ds(off, 16)] = vals
        return carry
      lax.fori_loop(0, GPC * GPC, tr_body, 0)
    pltpu.sync_copy(tiles_v.at[0], out_hbm.at[c, 0].at[pl.ds(bt0, GPC)])
    pltpu.sync_copy(tiles_v.at[1], out_hbm.at[c, 1].at[pl.ds(bt0, GPC)])
    return carry

  lax.fori_loop(0, NCHUNK, chunk_body, 0)


@jax.jit
def _embed(idx3d, flat_tab):
  mesh = plsc.VectorSubcoreMesh(core_axis_name="c", subcore_axis_name="s")
  f = pl.kernel(
      _gather_body,
      mesh=mesh,
      out_type=jax.ShapeDtypeStruct((N_COLS, 2, BT, 8, 128), jnp.float32),
      scratch_types=[
          pltpu.VMEM((GPC, 128), jnp.int32),
          pltpu.VMEM((CHUNK, DIM), jnp.float32),
          pltpu.VMEM((2, GPC, 8, 128), jnp.float32),
          pltpu.VMEM((128,), jnp.int32),
          pltpu.SemaphoreType.DMA,
      ],
      compiler_params=pltpu.CompilerParams(
          use_tc_tiling_on_sc=False, needs_layout_passes=False),
  )
  return f(idx3d, flat_tab)


def kernel(value, tables):
  flat_tab = tables.reshape(N_COLS * VOCAB, DIM)
  offs = jnp.arange(N_COLS, dtype=jnp.int32)[:, None] * VOCAB
  idx3d = (value.astype(jnp.int32).T + offs).reshape(N_COLS, BT, 128)
  out5d = _embed(idx3d, flat_tab)
  return out5d.transpose(2, 4, 0, 1, 3).reshape(B, N_COLS, DIM)


# speedup vs baseline: 1.3072x; 1.0859x over previous
import functools

import jax
import jax.numpy as jnp
from jax import lax
from jax.experimental import pallas as pl
from jax.experimental.pallas import tpu as pltpu
from jax.experimental.pallas import tpu_sc as plsc

B = 16384
N_COLS = 26
VOCAB = 100000
DIM = 16

NUM_CORES = 2
NUM_SUBCORES = 16
NW = NUM_CORES * NUM_SUBCORES
BT = B // 128
NTASK = N_COLS * BT
TPW = NTASK // NW
GPC = 8
CHUNK = GPC * 128
NCHUNK = TPW // GPC


def _gather_body(idx_hbm, tab_hbm, out_hbm, idx_v, rows_v, tiles_v, sem):
  wid = lax.axis_index("s") * NUM_CORES + lax.axis_index("c")
  t0 = wid * TPW

  base16 = lax.iota(jnp.int32, 16)
  dvec = [jnp.full((16,), d, jnp.int32) for d in range(DIM)]

  def chunk_body(k, carry):
    t = t0 + k * GPC
    c = t // BT
    bt0 = pl.multiple_of(t - c * BT, GPC)
    pltpu.sync_copy(idx_hbm.at[c].at[pl.ds(bt0, GPC)], idx_v)
    copies = []
    for j in range(GPC):
      copies.append(
          pltpu.async_copy(
              tab_hbm.at[idx_v.at[j]], rows_v.at[pl.ds(j * 128, 128)], sem))
    for cp in copies:
      cp.wait()
    for j in range(GPC):
      for d in range(DIM):
        dt, di = d // 8, d % 8
        for l in range(8):
          ridx = base16 + (j * 128 + l * 16)
          vals = plsc.load_gather(rows_v, [ridx, dvec[d]])
          tiles_v[dt, j, di, pl.ds(l * 16, 16)] = vals
    pltpu.sync_copy(tiles_v.at[0], out_hbm.at[c, 0].at[pl.ds(bt0, GPC)])
    pltpu.sync_copy(tiles_v.at[1], out_hbm.at[c, 1].at[pl.ds(bt0, GPC)])
    return carry

  lax.fori_loop(0, NCHUNK, chunk_body, 0)


@jax.jit
def _embed(idx3d, flat_tab):
  mesh = plsc.VectorSubcoreMesh(core_axis_name="c", subcore_axis_name="s")
  f = pl.kernel(
      _gather_body,
      mesh=mesh,
      out_type=jax.ShapeDtypeStruct((N_COLS, 2, BT, 8, 128), jnp.float32),
      scratch_types=[
          pltpu.VMEM((GPC, 128), jnp.int32),
          pltpu.VMEM((CHUNK, DIM), jnp.float32),
          pltpu.VMEM((2, GPC, 8, 128), jnp.float32),
          pltpu.SemaphoreType.DMA,
      ],
      compiler_params=pltpu.CompilerParams(
          use_tc_tiling_on_sc=False, needs_layout_passes=False),
  )
  return f(idx3d, flat_tab)


def kernel(value, tables):
  flat_tab = tables.reshape(N_COLS * VOCAB, DIM)
  offs = jnp.arange(N_COLS, dtype=jnp.int32)[:, None] * VOCAB
  idx3d = (value.astype(jnp.int32).T + offs).reshape(N_COLS, BT, 128)
  out5d = _embed(idx3d, flat_tab)
  return out5d.transpose(2, 4, 0, 1, 3).reshape(B, N_COLS, DIM)
